# baseline (speedup 1.0000x reference)
import functools

import jax
import jax.numpy as jnp
from jax import lax
from jax.experimental import pallas as pl
from jax.experimental.pallas import tpu as pltpu
from jax.experimental.pallas import tpu_sc as plsc

D = 64
B = 16384
E = 1000000
R = 1000
NC = 2
NS = 16
NW = NC * NS
BPW = B // NW
CHID = 512
NCH = E // CHID
TAIL0 = NCH * CHID
WIN = 512
DUMP = B
MROWS = B + 8
CEV = 80


def _make_k2():
    mesh = plsc.VectorSubcoreMesh(core_axis_name="c", subcore_axis_name="s")

    @functools.partial(
        pl.kernel,
        mesh=mesh,
        out_type=tuple(jax.ShapeDtypeStruct((MROWS, 128), jnp.float32)
                       for _ in range(4)),
        compiler_params=pltpu.CompilerParams(needs_layout_passes=False),
        scratch_types=[
            [pltpu.VMEM((8, 4, 8, 128), jnp.float32) for _ in range(2)],
            pltpu.VMEM((D, 64), jnp.float32),
            pltpu.VMEM((WIN,), jnp.int32),
            pltpu.VMEM((WIN,), jnp.int32),
            pltpu.VMEM((CEV,), jnp.int32),
            pltpu.VMEM((128, 128), jnp.float32),
            pltpu.VMEM((128,), jnp.int32),
            pltpu.VMEM((128,), jnp.int32),
            pltpu.SemaphoreType.DMA,
            pltpu.SemaphoreType.DMA,
            pltpu.SemaphoreType.DMA,
        ],
    )
    def k2(ent_r, ent_i, ids_s, tags_s, cbnd, hr_m, hi_m, tr_m, ti_m,
           blocks, tailblk, widv, wtagv, cev, stagev, ivh, ivt,
           sem0, sem1, sem2):
        wid = lax.axis_index("s") * NC + lax.axis_index("c")
        clo = wid * NCH // NW
        chi = (wid + 1) * NCH // NW
        lane = lax.iota(jnp.int32, 16)
        sems = (sem0, sem1)

        cb0 = pl.multiple_of((clo // 8) * 8, 8)
        pltpu.sync_copy(cbnd.at[pl.ds(cb0, CEV)], cev)

        def bscal(i):
            return plsc.load_gather(cev, [jnp.full((16,), i, jnp.int32)])[0]

        def reset_idx():
            dumpv = jnp.full((16,), DUMP, jnp.int32)
            for g in range(8):
                ivh[pl.ds(g * 16, 16)] = dumpv
                ivt[pl.ds(g * 16, 16)] = dumpv

        def flush(out_a, out_b):
            pltpu.async_copy(stagev, out_a.at[ivh], sem2)
            pltpu.async_copy(stagev, out_b.at[ivt], sem2)
            pltpu.make_async_copy(stagev, out_a.at[ivh], sem2).wait()
            pltpu.make_async_copy(stagev, out_b.at[ivt], sem2).wait()
            reset_idx()

        def extract_group(gi, carry, lo, width, gather1, out_a, out_b):
            wb, cnt = carry
            need = (gi * 16 < wb) | (gi * 16 + 16 > wb + WIN)
            nwb = pl.multiple_of(jnp.where(need, gi * 16, wb), 8)

            @pl.when(nwb != wb)
            def _():
                pltpu.sync_copy(ids_s.at[pl.ds(nwb, WIN)], widv)
                pltpu.sync_copy(tags_s.at[pl.ds(nwb, WIN)], wtagv)

            o = pl.multiple_of(gi * 16 - nwb, 8)
            idsv = widv[pl.ds(o, 16)]
            tagv = wtagv[pl.ds(o, 16)]
            m = (idsv >= lo) & (idsv < lo + width)
            lc = idsv - lo

            def dstep(d4, _):
                for q in range(4):
                    d = d4 * 4 + q
                    v = gather1(d, lc, m)
                    plsc.store_scatter(stagev,
                                       [cnt + lane,
                                        jnp.full((16,), d, jnp.int32)],
                                       v, mask=m)
                return 0

            lax.fori_loop(0, D // 4, dstep, 0)
            ivh[pl.ds(cnt, 16)] = jnp.where(m & (tagv < B), tagv, DUMP)
            ivt[pl.ds(cnt, 16)] = jnp.where(m & (tagv >= B), tagv - B, DUMP)
            ncnt = cnt + 16

            @pl.when(ncnt >= 128)
            def _():
                flush(out_a, out_b)

            return nwb, jnp.where(ncnt >= 128, 0, ncnt)

        def pass_one(tab, out_a, out_b):
            def tiles(c, slot):
                co = pl.multiple_of(c * CHID, 128)
                for i_s in range(8):
                    for j in range(4):
                        yield (tab.at[pl.ds(i_s * 8, 8),
                                      pl.ds(co + j * 128, 128)],
                               blocks[slot].at[i_s, j])

            def fire(c, slot):
                for src, dst in tiles(c, slot):
                    pltpu.async_copy(src, dst, sems[slot])

            def drain(c, slot):
                for src, dst in tiles(c, slot):
                    pltpu.make_async_copy(src, dst, sems[slot]).wait()

            def bgather(slot):
                def gather1(d, lc, m):
                    i16 = jnp.full((16,), d // 8, jnp.int32)
                    s16 = jnp.full((16,), d % 8, jnp.int32)
                    return plsc.load_gather(
                        blocks[slot], [i16, lc >> 7, s16, lc & 127], mask=m)
                return gather1

            def process(c, slot, carry):
                ja = bscal(c - cb0)
                jb = bscal(c - cb0 + 1)

                def grp(gi, carry):
                    return extract_group(gi, carry, c * CHID, CHID,
                                         bgather(slot), out_a, out_b)

                ghi = jnp.where(jb > ja, (jb + 15) // 16, ja // 16)
                return lax.fori_loop(ja // 16, ghi, grp, carry)

            nsteps = (chi - clo + 1) // 2
            reset_idx()
            fire(clo, 0)

            def step(i, carry):
                c = clo + i * 2

                @pl.when(c + 1 < chi)
                def _():
                    fire(c + 1, 1)

                drain(c, 0)
                carry = process(c, 0, carry)

                @pl.when(c + 2 < chi)
                def _():
                    fire(c + 2, 0)

                def odd(carry):
                    drain(c + 1, 1)
                    return process(c + 1, 1, carry)

                return lax.cond(c + 1 < chi, odd, lambda x: x, carry)

            def gstep(i, carry):
                return lax.cond(i < nsteps, lambda x: step(i, x),
                                lambda x: x, carry)

            carry = lax.fori_loop(0, NCH // NW // 2 + 2, gstep,
                                  (jnp.int32(-WIN), jnp.int32(0)))

            def tail_pass(carry):
                pltpu.sync_copy(tab.at[:, pl.ds(TAIL0, 64)], tailblk)
                ja = bscal(NCH - cb0)
                jb = bscal(NCH + 1 - cb0)

                def tgather(d, lc, m):
                    return plsc.load_gather(
                        tailblk, [jnp.full((16,), d, jnp.int32), lc], mask=m)

                def tgrp(gi, carry):
                    return extract_group(gi, carry, TAIL0, 64, tgather,
                                         out_a, out_b)

                ghi = jnp.where(jb > ja, (jb + 15) // 16, ja // 16)
                return lax.fori_loop(ja // 16, ghi, tgrp, carry)

            carry = lax.cond(wid == NW - 1, tail_pass, lambda x: x, carry)
            flush(out_a, out_b)

        pass_one(ent_r, hr_m, tr_m)
        pass_one(ent_i, hi_m, ti_m)

    return k2


def _make_k3():
    mesh = plsc.VectorSubcoreMesh(core_axis_name="c", subcore_axis_name="s")

    @functools.partial(
        pl.kernel,
        mesh=mesh,
        out_type=jax.ShapeDtypeStruct((B,), jnp.float32),
        compiler_params=pltpu.CompilerParams(needs_layout_passes=False),
        scratch_types=[
            pltpu.VMEM((8, R), jnp.float32),
            pltpu.VMEM((8, R), jnp.float32),
            pltpu.VMEM((BPW * D,), jnp.float32),
            pltpu.VMEM((BPW * D,), jnp.float32),
            pltpu.VMEM((BPW,), jnp.int32),
            [pltpu.VMEM((64, 128), jnp.float32) for _ in range(4)],
            pltpu.VMEM((64 * 16,), jnp.float32),
            pltpu.VMEM((BPW,), jnp.float32),
            pltpu.SemaphoreType.DMA,
        ],
    )
    def k3(relation, rel_r, rel_i, hr_m, hi_m, tr_m, ti_m, out,
           stripr, stripi, rrow_r, rrow_i, qv, mats, stage, out_v, sem):
        wid = lax.axis_index("s") * NC + lax.axis_index("c")
        base = pl.multiple_of(wid * BPW, 8)
        lane = lax.iota(jnp.int32, 16)
        lane16 = lane * 16
        pltpu.sync_copy(relation.at[pl.ds(base, BPW)], qv)

        for i_strip in range(8):
            pltpu.sync_copy(rel_r.at[pl.ds(i_strip * 8, 8), :], stripr)
            pltpu.sync_copy(rel_i.at[pl.ds(i_strip * 8, 8), :], stripi)

            def sg(g, _, i_strip=i_strip):
                q16 = qv[pl.ds(g * 16, 16)]
                for s in range(8):
                    d = i_strip * 8 + s
                    sv = jnp.full((16,), s, jnp.int32)
                    vr = plsc.load_gather(stripr, [sv, q16])
                    vi = plsc.load_gather(stripi, [sv, q16])
                    idx = g * 16 * D + lane * D + d
                    plsc.store_scatter(rrow_r, [idx], vr)
                    plsc.store_scatter(rrow_i, [idx], vi)
                return 0

            lax.fori_loop(0, BPW // 16, sg, 0)

        def chunk(ch, _):
            off = pl.multiple_of(base + ch * 64, 8)
            pltpu.sync_copy(hr_m.at[pl.ds(off, 64), :], mats[0])
            pltpu.sync_copy(hi_m.at[pl.ds(off, 64), :], mats[1])
            pltpu.sync_copy(tr_m.at[pl.ds(off, 64), :], mats[2])
            pltpu.sync_copy(ti_m.at[pl.ds(off, 64), :], mats[3])

            def row(u, _):
                i = ch * 64 + u
                acc = jnp.zeros((16,), jnp.float32)
                for k in range(D // 16):
                    s = pl.ds(k * 16, 16)
                    hrv = mats[0][u, s]
                    hiv = mats[1][u, s]
                    trv = mats[2][u, s]
                    tiv = mats[3][u, s]
                    rrv = rrow_r[pl.ds(i * D + k * 16, 16)]
                    riv = rrow_i[pl.ds(i * D + k * 16, 16)]
                    a = hrv * trv - hiv * tiv
                    bb = hrv * tiv + hiv * trv
                    acc = acc + rrv * a + riv * bb
                stage[pl.ds(u * 16, 16)] = acc
                return 0

            lax.fori_loop(0, 64, row, 0)

            def group(g, _):
                gbase = g * 256
                tot = jnp.zeros((16,), jnp.float32)
                for j in range(16):
                    tot = tot + plsc.load_gather(stage, [gbase + lane16 + j])
                out_v[pl.ds(ch * 64 + g * 16, 16)] = tot
                return 0

            lax.fori_loop(0, 4, group, 0)
            return 0

        lax.fori_loop(0, BPW // 64, chunk, 0)
        pltpu.sync_copy(out_v, out.at[pl.ds(base, BPW)])

    return k3


_K2 = _make_k2()
_K3 = _make_k3()


def kernel(head, relation, tail, entity_real, entity_imag,
           relation_real, relation_imag):
    ids = jnp.concatenate([head, tail])
    order = jnp.arange(2 * B, dtype=jnp.int32)
    ids_s, tags_s = lax.sort([ids, order], num_keys=1)
    edges = jnp.arange(NCH + 2, dtype=jnp.int32) * CHID
    cbnd = jnp.searchsorted(ids_s, edges).astype(jnp.int32)
    cbnd = jnp.pad(cbnd, (0, CEV))
    ids_pad = jnp.pad(ids_s, (0, WIN))
    tags_pad = jnp.pad(tags_s, (0, WIN))
    hr_m, hi_m, tr_m, ti_m = _K2(entity_real.T, entity_imag.T,
                                 ids_pad, tags_pad, cbnd)
    return _K3(relation, relation_real.T, relation_imag.T,
               hr_m, hi_m, tr_m, ti_m)


# speedup vs baseline: 4.3659x; 4.3659x over previous
import functools

import jax
import jax.numpy as jnp
from jax import lax
from jax.experimental import pallas as pl
from jax.experimental.pallas import tpu as pltpu
from jax.experimental.pallas import tpu_sc as plsc

D = 64
DP = 128
B = 16384
NC = 2
NS = 16
NW = NC * NS
BPW = B // NW
C = 64
NCHUNK = BPW // C


def _make_kernel():
    mesh = plsc.VectorSubcoreMesh(core_axis_name="c", subcore_axis_name="s")

    buf = lambda: pltpu.VMEM((C, DP), jnp.float32)

    @functools.partial(
        pl.kernel,
        mesh=mesh,
        out_type=jax.ShapeDtypeStruct((B,), jnp.float32),
        compiler_params=pltpu.CompilerParams(needs_layout_passes=False),
        scratch_types=[
            [pltpu.VMEM((C,), jnp.int32) for _ in range(2)],
            [pltpu.VMEM((C,), jnp.int32) for _ in range(2)],
            [pltpu.VMEM((C,), jnp.int32) for _ in range(2)],
            [[pltpu.VMEM((C, DP), jnp.float32) for _ in range(6)]
             for _ in range(2)],
            pltpu.VMEM((C * 16,), jnp.float32),
            pltpu.VMEM((BPW,), jnp.float32),
            pltpu.SemaphoreType.DMA,
            pltpu.SemaphoreType.DMA,
        ],
    )
    def complex_score(head, relation, tail, ent_r, ent_i, rel_r, rel_i,
                      out, ih, ir, it, bufs, stage, out_v, sem0, sem1):
        wid = lax.axis_index("s") * NC + lax.axis_index("c")
        base = wid * BPW
        sems = (sem0, sem1)
        lane16 = lax.iota(jnp.int32, 16) * 16

        def load_idx(c, slot):
            off = pl.multiple_of(base + c * C, 8)
            pltpu.sync_copy(head.at[pl.ds(off, C)], ih[slot])
            pltpu.sync_copy(relation.at[pl.ds(off, C)], ir[slot])
            pltpu.sync_copy(tail.at[pl.ds(off, C)], it[slot])

        def srcs(slot):
            return ((ent_r.at[ih[slot]], 0), (ent_i.at[ih[slot]], 1),
                    (ent_r.at[it[slot]], 2), (ent_i.at[it[slot]], 3),
                    (rel_r.at[ir[slot]], 4), (rel_i.at[ir[slot]], 5))

        def fire(slot):
            for src, t in srcs(slot):
                pltpu.async_copy(src, bufs[slot][t], sems[slot])

        def drain(slot):
            for src, t in srcs(slot):
                pltpu.make_async_copy(src, bufs[slot][t], sems[slot]).wait()

        def compute(slot):
            def row(i, _):
                acc = jnp.zeros((16,), jnp.float32)
                for k in range(D // 16):
                    s = pl.ds(k * 16, 16)
                    hrv = bufs[slot][0][i, s]
                    hiv = bufs[slot][1][i, s]
                    trv = bufs[slot][2][i, s]
                    tiv = bufs[slot][3][i, s]
                    rrv = bufs[slot][4][i, s]
                    riv = bufs[slot][5][i, s]
                    a = hrv * trv - hiv * tiv
                    bb = hrv * tiv + hiv * trv
                    acc = acc + rrv * a + riv * bb
                stage[pl.ds(i * 16, 16)] = acc
                return 0

            lax.fori_loop(0, C, row, 0)

        def reduce_out(c):
            def group(g, _):
                gbase = g * 256
                tot = jnp.zeros((16,), jnp.float32)
                for j in range(16):
                    tot = tot + plsc.load_gather(stage,
                                                 [gbase + lane16 + j])
                out_v[pl.ds(c * C + g * 16, 16)] = tot
                return 0

            lax.fori_loop(0, C // 16, group, 0)

        load_idx(0, 0)
        fire(0)
        for c in range(NCHUNK):
            slot = c % 2
            if c + 1 < NCHUNK:
                load_idx(c + 1, 1 - slot)
                fire(1 - slot)
            drain(slot)
            compute(slot)
            reduce_out(c)
        pltpu.sync_copy(out_v, out.at[pl.ds(base, BPW)])

    return complex_score


_KERNEL = _make_kernel()


def kernel(head, relation, tail, entity_real, entity_imag,
           relation_real, relation_imag):
    pad = ((0, 0), (0, DP - D))
    ent_r = jnp.pad(entity_real, pad)
    ent_i = jnp.pad(entity_imag, pad)
    rel_r = jnp.pad(relation_real, pad)
    rel_i = jnp.pad(relation_imag, pad)
    return _KERNEL(head, relation, tail, ent_r, ent_i, rel_r, rel_i)
